# baseline (speedup 1.0000x reference)
import jax
import jax.numpy as jnp
from jax.experimental import pallas as pl
from jax.experimental.pallas import tpu as pltpu

BS, C, H, W = 16, 512, 64, 64
HW = H * W
CT = 128
NCT = C // CT
KP = 64


def _fused_body(query_ref, wq_ref, p_ref, w2_ref, bias_ref, wl_ref, bl_ref,
                key_ref, value_ref, layouts_ref, km_ref, cum_ref,
                logit_ref, saw_ref, out_ref, cumo_ref):
    ct = pl.program_id(1)

    qv = jax.lax.dot_general(wq_ref[...], query_ref[0],
                             (((1,), (1,)), ((), ())),
                             preferred_element_type=jnp.float32)
    conv = jax.lax.dot_general(w2_ref[...], p_ref[0],
                               (((1,), (0,)), ((), ())),
                               preferred_element_type=jnp.float32)
    fusion = key_ref[0] + conv + qv + bias_ref[...]
    t = jnp.tanh(fusion)
    part = jax.lax.dot_general(wl_ref[...], t, (((1,), (0,)), ((), ())),
                               preferred_element_type=jnp.float32)

    maskf = (layouts_ref[0] == 1.0).astype(jnp.float32)
    psum = jax.lax.dot_general(maskf, value_ref[0],
                               (((1,), (1,)), ((), ())),
                               preferred_element_type=jnp.float32)
    cnt = jnp.sum(maskf)
    out_ref[0] = jnp.where(cnt > 0.0, psum / jnp.maximum(cnt, 1.0), 0.0)

    @pl.when(ct == 0)
    def _():
        logit_ref[0] = part + bl_ref[0, 0]
        cumo_ref[0] = jnp.minimum(layouts_ref[0] + cum_ref[0], 1.0)

    @pl.when(ct != 0)
    def _():
        logit_ref[0] += part

    @pl.when(ct == NCT - 1)
    def _():
        l = logit_ref[0] - (1.0 - km_ref[0]) * 100000000.0
        m = jnp.max(l, axis=1, keepdims=True)
        e = jnp.exp(l - m)
        saw_ref[0] = e / jnp.sum(e, axis=1, keepdims=True)


def _im2col(x):
    xp = jnp.pad(x, ((0, 0), (2, 2), (2, 2)))
    cols = [xp[:, dy:dy + H, dx:dx + W].reshape(BS, 1, HW)
            for dy in range(5) for dx in range(5)]
    return jnp.concatenate(cols, axis=1)


def kernel(key, key_mask, query, spatial_att_weight, cum_spatial_att_weight,
           value, state, layouts, Wq, bq, Ww, bw, Wc, bc, Wl, bl):
    key_r = key.reshape(BS, C, HW)
    value_r = value.reshape(BS, C, HW)
    km = key_mask.reshape(BS, 1, HW)
    cum = cum_spatial_att_weight.reshape(BS, 1, HW)
    lay = layouts.reshape(BS, 1, HW)
    query_r = query.reshape(BS, 1, C)

    p1 = _im2col(spatial_att_weight.reshape(BS, H, W))
    p2 = _im2col(cum_spatial_att_weight.reshape(BS, H, W))
    pz = jnp.zeros((BS, KP - 50, HW), jnp.float32)
    P = jnp.concatenate([p1, p2, pz], axis=1)
    W2 = jnp.concatenate([Ww.reshape(C, 25), Wc.reshape(C, 25),
                          jnp.zeros((C, KP - 50), jnp.float32)], axis=1)
    bias = (bq + bw + bc).reshape(C, 1)
    Wl2 = Wl.reshape(1, C)

    grid = (BS, NCT)
    out_shape = [
        jax.ShapeDtypeStruct((BS, 1, HW), jnp.float32),
        jax.ShapeDtypeStruct((BS, 1, HW), jnp.float32),
        jax.ShapeDtypeStruct((BS, 1, C), jnp.float32),
        jax.ShapeDtypeStruct((BS, 1, HW), jnp.float32),
    ]
    in_specs = [
        pl.BlockSpec((1, 1, C), lambda b, c: (b, 0, 0)),
        pl.BlockSpec((CT, C), lambda b, c: (c, 0)),
        pl.BlockSpec((1, KP, HW), lambda b, c: (b, 0, 0)),
        pl.BlockSpec((CT, KP), lambda b, c: (c, 0)),
        pl.BlockSpec((CT, 1), lambda b, c: (c, 0)),
        pl.BlockSpec((1, CT), lambda b, c: (0, c)),
        pl.BlockSpec((1, 1), lambda b, c: (0, 0)),
        pl.BlockSpec((1, CT, HW), lambda b, c: (b, c, 0)),
        pl.BlockSpec((1, CT, HW), lambda b, c: (b, c, 0)),
        pl.BlockSpec((1, 1, HW), lambda b, c: (b, 0, 0)),
        pl.BlockSpec((1, 1, HW), lambda b, c: (b, 0, 0)),
        pl.BlockSpec((1, 1, HW), lambda b, c: (b, 0, 0)),
    ]
    out_specs = [
        pl.BlockSpec((1, 1, HW), lambda b, c: (b, 0, 0)),
        pl.BlockSpec((1, 1, HW), lambda b, c: (b, 0, 0)),
        pl.BlockSpec((1, 1, CT), lambda b, c: (b, 0, c)),
        pl.BlockSpec((1, 1, HW), lambda b, c: (b, 0, 0)),
    ]
    logit, saw, outputs, cumo = pl.pallas_call(
        _fused_body,
        grid=grid,
        in_specs=in_specs,
        out_specs=out_specs,
        out_shape=out_shape,
        compiler_params=pltpu.CompilerParams(
            dimension_semantics=("arbitrary", "arbitrary")),
    )(query_r, Wq, P, W2, bias, Wl2, bl.reshape(1, 1), key_r, value_r,
      lay, km, cum)

    return (state,
            outputs.reshape(BS, C),
            logit.reshape(BS, 1, H, W),
            saw.reshape(BS, 1, H, W),
            cumo.reshape(BS, 1, H, W))


# speedup vs baseline: 1.0747x; 1.0747x over previous
import jax
import jax.numpy as jnp
from jax.experimental import pallas as pl
from jax.experimental.pallas import tpu as pltpu

BS, C, H, W = 16, 512, 64, 64
HW = H * W
KP = 64


def _fused_body(query_ref, wq_ref, p_ref, w2_ref, bias_ref, wl_ref, bl_ref,
                key_ref, value_ref, layouts_ref, km_ref, cum_ref,
                logit_ref, saw_ref, out_ref, cumo_ref):
    qv = jax.lax.dot_general(wq_ref[...], query_ref[0],
                             (((1,), (1,)), ((), ())),
                             preferred_element_type=jnp.float32)
    conv = jax.lax.dot_general(w2_ref[...], p_ref[0],
                               (((1,), (0,)), ((), ())),
                               preferred_element_type=jnp.float32)
    fusion = key_ref[0] + conv + qv + bias_ref[...]
    t = jnp.tanh(fusion)
    logit = jax.lax.dot_general(wl_ref[...], t, (((1,), (0,)), ((), ())),
                                preferred_element_type=jnp.float32) + bl_ref[0, 0]
    logit_ref[0] = logit

    l = logit - (1.0 - km_ref[0]) * 100000000.0
    m = jnp.max(l, axis=1, keepdims=True)
    e = jnp.exp(l - m)
    saw_ref[0] = e / jnp.sum(e, axis=1, keepdims=True)

    maskf = (layouts_ref[0] == 1.0).astype(jnp.float32)
    psum = jax.lax.dot_general(maskf, value_ref[0],
                               (((1,), (1,)), ((), ())),
                               preferred_element_type=jnp.float32)
    cnt = jnp.sum(maskf)
    out_ref[0] = jnp.where(cnt > 0.0, psum / jnp.maximum(cnt, 1.0), 0.0)

    cumo_ref[0] = jnp.minimum(layouts_ref[0] + cum_ref[0], 1.0)


def _im2col(x):
    xp = jnp.pad(x, ((0, 0), (2, 2), (2, 2)))
    cols = [xp[:, dy:dy + H, dx:dx + W].reshape(BS, 1, HW)
            for dy in range(5) for dx in range(5)]
    return jnp.concatenate(cols, axis=1)


def kernel(key, key_mask, query, spatial_att_weight, cum_spatial_att_weight,
           value, state, layouts, Wq, bq, Ww, bw, Wc, bc, Wl, bl):
    key_r = key.reshape(BS, C, HW)
    value_r = value.reshape(BS, C, HW)
    km = key_mask.reshape(BS, 1, HW)
    cum = cum_spatial_att_weight.reshape(BS, 1, HW)
    lay = layouts.reshape(BS, 1, HW)
    query_r = query.reshape(BS, 1, C)

    p1 = _im2col(spatial_att_weight.reshape(BS, H, W))
    p2 = _im2col(cum_spatial_att_weight.reshape(BS, H, W))
    pz = jnp.zeros((BS, KP - 50, HW), jnp.float32)
    P = jnp.concatenate([p1, p2, pz], axis=1)
    W2 = jnp.concatenate([Ww.reshape(C, 25), Wc.reshape(C, 25),
                          jnp.zeros((C, KP - 50), jnp.float32)], axis=1)
    bias = (bq + bw + bc).reshape(C, 1)
    Wl2 = Wl.reshape(1, C)

    grid = (BS,)
    out_shape = [
        jax.ShapeDtypeStruct((BS, 1, HW), jnp.float32),
        jax.ShapeDtypeStruct((BS, 1, HW), jnp.float32),
        jax.ShapeDtypeStruct((BS, 1, C), jnp.float32),
        jax.ShapeDtypeStruct((BS, 1, HW), jnp.float32),
    ]
    in_specs = [
        pl.BlockSpec((1, 1, C), lambda b: (b, 0, 0)),
        pl.BlockSpec((C, C), lambda b: (0, 0)),
        pl.BlockSpec((1, KP, HW), lambda b: (b, 0, 0)),
        pl.BlockSpec((C, KP), lambda b: (0, 0)),
        pl.BlockSpec((C, 1), lambda b: (0, 0)),
        pl.BlockSpec((1, C), lambda b: (0, 0)),
        pl.BlockSpec((1, 1), lambda b: (0, 0)),
        pl.BlockSpec((1, C, HW), lambda b: (b, 0, 0)),
        pl.BlockSpec((1, C, HW), lambda b: (b, 0, 0)),
        pl.BlockSpec((1, 1, HW), lambda b: (b, 0, 0)),
        pl.BlockSpec((1, 1, HW), lambda b: (b, 0, 0)),
        pl.BlockSpec((1, 1, HW), lambda b: (b, 0, 0)),
    ]
    out_specs = [
        pl.BlockSpec((1, 1, HW), lambda b: (b, 0, 0)),
        pl.BlockSpec((1, 1, HW), lambda b: (b, 0, 0)),
        pl.BlockSpec((1, 1, C), lambda b: (b, 0, 0)),
        pl.BlockSpec((1, 1, HW), lambda b: (b, 0, 0)),
    ]
    logit, saw, outputs, cumo = pl.pallas_call(
        _fused_body,
        grid=grid,
        in_specs=in_specs,
        out_specs=out_specs,
        out_shape=out_shape,
        compiler_params=pltpu.CompilerParams(
            dimension_semantics=("arbitrary",)),
    )(query_r, Wq, P, W2, bias, Wl2, bl.reshape(1, 1), key_r, value_r,
      lay, km, cum)

    return (state,
            outputs.reshape(BS, C),
            logit.reshape(BS, 1, H, W),
            saw.reshape(BS, 1, H, W),
            cumo.reshape(BS, 1, H, W))


# speedup vs baseline: 1.6146x; 1.5023x over previous
import jax
import jax.numpy as jnp
from jax.experimental import pallas as pl
from jax.experimental.pallas import tpu as pltpu

BS, C, H, W = 16, 512, 64, 64
HW = H * W
KP = 64


def _fused_body(query_ref, wq_ref, saw_ref, w2_ref, bias_ref, wl_ref, bl_ref,
                key_ref, value_ref, layouts_ref, km_ref, cum_ref,
                logit_ref, sawo_ref, out_ref, cumo_ref, p_scr):
    pos = jax.lax.broadcasted_iota(jnp.int32, (1, HW), 1)
    hh = pos // W
    ww = pos % W
    mh = {d: ((hh + d >= 0) & (hh + d < H)).astype(jnp.float32)
          for d in range(-2, 3)}
    mw = {d: ((ww + d >= 0) & (ww + d < W)).astype(jnp.float32)
          for d in range(-2, 3)}
    k = 0
    for src_ref in (saw_ref, cum_ref):
        src = src_ref[0]
        for dy in range(-2, 3):
            for dx in range(-2, 3):
                s = dy * W + dx
                rolled = src if s == 0 else jnp.roll(src, -s, axis=1)
                p_scr[k:k + 1, :] = rolled * mh[dy] * mw[dx]
                k += 1
    p_scr[50:KP, :] = jnp.zeros((KP - 50, HW), jnp.float32)

    qv = jax.lax.dot_general(wq_ref[...], query_ref[0],
                             (((1,), (1,)), ((), ())),
                             preferred_element_type=jnp.float32)
    conv = jax.lax.dot_general(w2_ref[...], p_scr[...],
                               (((1,), (0,)), ((), ())),
                               preferred_element_type=jnp.float32)
    fusion = key_ref[0] + conv + qv + bias_ref[...]
    t = jnp.tanh(fusion)
    logit = jax.lax.dot_general(wl_ref[...], t, (((1,), (0,)), ((), ())),
                                preferred_element_type=jnp.float32) + bl_ref[0, 0]
    logit_ref[0] = logit

    l = logit - (1.0 - km_ref[0]) * 100000000.0
    m = jnp.max(l, axis=1, keepdims=True)
    e = jnp.exp(l - m)
    sawo_ref[0] = e / jnp.sum(e, axis=1, keepdims=True)

    maskf = (layouts_ref[0] == 1.0).astype(jnp.float32)
    psum = jax.lax.dot_general(maskf, value_ref[0],
                               (((1,), (1,)), ((), ())),
                               preferred_element_type=jnp.float32)
    cnt = jnp.sum(maskf)
    out_ref[0] = jnp.where(cnt > 0.0, psum / jnp.maximum(cnt, 1.0), 0.0)

    cumo_ref[0] = jnp.minimum(layouts_ref[0] + cum_ref[0], 1.0)


def kernel(key, key_mask, query, spatial_att_weight, cum_spatial_att_weight,
           value, state, layouts, Wq, bq, Ww, bw, Wc, bc, Wl, bl):
    key_r = key.reshape(BS, C, HW)
    value_r = value.reshape(BS, C, HW)
    km = key_mask.reshape(BS, 1, HW)
    saw = spatial_att_weight.reshape(BS, 1, HW)
    cum = cum_spatial_att_weight.reshape(BS, 1, HW)
    lay = layouts.reshape(BS, 1, HW)
    query_r = query.reshape(BS, 1, C)

    W2 = jnp.concatenate([Ww.reshape(C, 25), Wc.reshape(C, 25),
                          jnp.zeros((C, KP - 50), jnp.float32)], axis=1)
    bias = (bq + bw + bc).reshape(C, 1)
    Wl2 = Wl.reshape(1, C)

    grid = (BS,)
    out_shape = [
        jax.ShapeDtypeStruct((BS, 1, HW), jnp.float32),
        jax.ShapeDtypeStruct((BS, 1, HW), jnp.float32),
        jax.ShapeDtypeStruct((BS, 1, C), jnp.float32),
        jax.ShapeDtypeStruct((BS, 1, HW), jnp.float32),
    ]
    in_specs = [
        pl.BlockSpec((1, 1, C), lambda b: (b, 0, 0)),
        pl.BlockSpec((C, C), lambda b: (0, 0)),
        pl.BlockSpec((1, 1, HW), lambda b: (b, 0, 0)),
        pl.BlockSpec((C, KP), lambda b: (0, 0)),
        pl.BlockSpec((C, 1), lambda b: (0, 0)),
        pl.BlockSpec((1, C), lambda b: (0, 0)),
        pl.BlockSpec((1, 1), lambda b: (0, 0)),
        pl.BlockSpec((1, C, HW), lambda b: (b, 0, 0)),
        pl.BlockSpec((1, C, HW), lambda b: (b, 0, 0)),
        pl.BlockSpec((1, 1, HW), lambda b: (b, 0, 0)),
        pl.BlockSpec((1, 1, HW), lambda b: (b, 0, 0)),
        pl.BlockSpec((1, 1, HW), lambda b: (b, 0, 0)),
    ]
    out_specs = [
        pl.BlockSpec((1, 1, HW), lambda b: (b, 0, 0)),
        pl.BlockSpec((1, 1, HW), lambda b: (b, 0, 0)),
        pl.BlockSpec((1, 1, C), lambda b: (b, 0, 0)),
        pl.BlockSpec((1, 1, HW), lambda b: (b, 0, 0)),
    ]
    logit, sawo, outputs, cumo = pl.pallas_call(
        _fused_body,
        grid=grid,
        in_specs=in_specs,
        out_specs=out_specs,
        out_shape=out_shape,
        scratch_shapes=[pltpu.VMEM((KP, HW), jnp.float32)],
        compiler_params=pltpu.CompilerParams(
            dimension_semantics=("arbitrary",)),
    )(query_r, Wq, saw, W2, bias, Wl2, bl.reshape(1, 1), key_r, value_r,
      lay, km, cum)

    return (state,
            outputs.reshape(BS, C),
            logit.reshape(BS, 1, H, W),
            sawo.reshape(BS, 1, H, W),
            cumo.reshape(BS, 1, H, W))


# speedup vs baseline: 1.6244x; 1.0061x over previous
import jax
import jax.numpy as jnp
from jax.experimental import pallas as pl
from jax.experimental.pallas import tpu as pltpu

BS, C, H, W = 16, 512, 64, 64
HW = H * W
CH = C // 2
KP = 64


def _fused_body(query_ref, wq_ref, saw_ref, w2_ref, bias_ref, wl_ref, bl_ref,
                key0_ref, key1_ref, val0_ref, val1_ref,
                layouts_ref, km_ref, cum_ref,
                logit_ref, sawo_ref, out_ref, cumo_ref, p_scr):
    pos = jax.lax.broadcasted_iota(jnp.int32, (1, HW), 1)
    hh = pos // W
    ww = pos % W
    mh = {d: ((hh + d >= 0) & (hh + d < H)).astype(jnp.float32)
          for d in range(-2, 3)}
    mw = {d: ((ww + d >= 0) & (ww + d < W)).astype(jnp.float32)
          for d in range(-2, 3)}
    k = 0
    for src_ref in (saw_ref, cum_ref):
        src = src_ref[0]
        for dy in range(-2, 3):
            for dx in range(-2, 3):
                s = dy * W + dx
                rolled = src if s == 0 else jnp.roll(src, -s, axis=1)
                p_scr[k:k + 1, :] = rolled * mh[dy] * mw[dx]
                k += 1
    p_scr[50:KP, :] = jnp.zeros((KP - 50, HW), jnp.float32)

    def half(key_ref, lo):
        sl = slice(lo, lo + CH)
        qv = jax.lax.dot_general(wq_ref[sl, :], query_ref[0],
                                 (((1,), (1,)), ((), ())),
                                 preferred_element_type=jnp.float32)
        conv = jax.lax.dot_general(w2_ref[sl, :], p_scr[...],
                                   (((1,), (0,)), ((), ())),
                                   preferred_element_type=jnp.float32)
        fusion = key_ref[0] + conv + qv + bias_ref[sl, :]
        t = jnp.tanh(fusion)
        return jax.lax.dot_general(wl_ref[:, sl], t, (((1,), (0,)), ((), ())),
                                   preferred_element_type=jnp.float32)

    logit = half(key0_ref, 0) + half(key1_ref, CH) + bl_ref[0, 0]
    logit_ref[0] = logit

    l = logit - (1.0 - km_ref[0]) * 100000000.0
    m = jnp.max(l, axis=1, keepdims=True)
    e = jnp.exp(l - m)
    sawo_ref[0] = e / jnp.sum(e, axis=1, keepdims=True)

    maskf = (layouts_ref[0] == 1.0).astype(jnp.float32)
    cnt = jnp.sum(maskf)
    scale = jnp.where(cnt > 0.0, 1.0 / jnp.maximum(cnt, 1.0), 0.0)
    for val_ref, lo in ((val0_ref, 0), (val1_ref, CH)):
        psum = jax.lax.dot_general(maskf, val_ref[0],
                                   (((1,), (1,)), ((), ())),
                                   preferred_element_type=jnp.float32)
        out_ref[0, :, lo:lo + CH] = psum * scale

    cumo_ref[0] = jnp.minimum(layouts_ref[0] + cum_ref[0], 1.0)


def kernel(key, key_mask, query, spatial_att_weight, cum_spatial_att_weight,
           value, state, layouts, Wq, bq, Ww, bw, Wc, bc, Wl, bl):
    key_r = key.reshape(BS, C, HW)
    value_r = value.reshape(BS, C, HW)
    km = key_mask.reshape(BS, 1, HW)
    saw = spatial_att_weight.reshape(BS, 1, HW)
    cum = cum_spatial_att_weight.reshape(BS, 1, HW)
    lay = layouts.reshape(BS, 1, HW)
    query_r = query.reshape(BS, 1, C)

    W2 = jnp.concatenate([Ww.reshape(C, 25), Wc.reshape(C, 25),
                          jnp.zeros((C, KP - 50), jnp.float32)], axis=1)
    bias = (bq + bw + bc).reshape(C, 1)
    Wl2 = Wl.reshape(1, C)

    grid = (BS,)
    out_shape = [
        jax.ShapeDtypeStruct((BS, 1, HW), jnp.float32),
        jax.ShapeDtypeStruct((BS, 1, HW), jnp.float32),
        jax.ShapeDtypeStruct((BS, 1, C), jnp.float32),
        jax.ShapeDtypeStruct((BS, 1, HW), jnp.float32),
    ]
    half_spec = pl.BlockSpec((1, CH, HW), lambda b, i=0: (b, 0, 0))
    in_specs = [
        pl.BlockSpec((1, 1, C), lambda b: (b, 0, 0)),
        pl.BlockSpec((C, C), lambda b: (0, 0)),
        pl.BlockSpec((1, 1, HW), lambda b: (b, 0, 0)),
        pl.BlockSpec((C, KP), lambda b: (0, 0)),
        pl.BlockSpec((C, 1), lambda b: (0, 0)),
        pl.BlockSpec((1, C), lambda b: (0, 0)),
        pl.BlockSpec((1, 1), lambda b: (0, 0)),
        pl.BlockSpec((1, CH, HW), lambda b: (b, 0, 0)),
        pl.BlockSpec((1, CH, HW), lambda b: (b, 1, 0)),
        pl.BlockSpec((1, CH, HW), lambda b: (b, 0, 0)),
        pl.BlockSpec((1, CH, HW), lambda b: (b, 1, 0)),
        pl.BlockSpec((1, 1, HW), lambda b: (b, 0, 0)),
        pl.BlockSpec((1, 1, HW), lambda b: (b, 0, 0)),
        pl.BlockSpec((1, 1, HW), lambda b: (b, 0, 0)),
    ]
    out_specs = [
        pl.BlockSpec((1, 1, HW), lambda b: (b, 0, 0)),
        pl.BlockSpec((1, 1, HW), lambda b: (b, 0, 0)),
        pl.BlockSpec((1, 1, C), lambda b: (b, 0, 0)),
        pl.BlockSpec((1, 1, HW), lambda b: (b, 0, 0)),
    ]
    logit, sawo, outputs, cumo = pl.pallas_call(
        _fused_body,
        grid=grid,
        in_specs=in_specs,
        out_specs=out_specs,
        out_shape=out_shape,
        scratch_shapes=[pltpu.VMEM((KP, HW), jnp.float32)],
        compiler_params=pltpu.CompilerParams(
            dimension_semantics=("arbitrary",)),
    )(query_r, Wq, saw, W2, bias, Wl2, bl.reshape(1, 1),
      key_r, key_r, value_r, value_r, lay, km, cum)

    return (state,
            outputs.reshape(BS, C),
            logit.reshape(BS, 1, H, W),
            sawo.reshape(BS, 1, H, W),
            cumo.reshape(BS, 1, H, W))
